# Initial kernel scaffold; baseline (speedup 1.0000x reference)
import functools

import jax
import jax.numpy as jnp
from jax import lax
from jax.experimental import pallas as pl
from jax.experimental.pallas import tpu as pltpu
from jax.experimental.pallas import tpu_sc as plsc

B = 16384
NNZ = 26
K = 32
NC = 2
NS = 16
NW = NC * NS
CB = 64
ROWS_PER_W = B // NW
CHUNKS = ROWS_PER_W // CB
G = CB * NNZ // 128


def _tower_body(u_hbm, v_hbm, wu_hbm, wv_hbm, p_hbm, q_hbm,
                idx_v, rows_v, out_v, sem):
    wid = lax.axis_index("s") * NC + lax.axis_index("c")

    def run(idx_hbm, tab_hbm, out_hbm, base_row):
        def chunk_body(ci, _):
            row0 = base_row + ci * CB
            irow = (row0 * NNZ) // 128
            pltpu.sync_copy(idx_hbm.at[pl.ds(irow, G)], idx_v)
            cps = [
                pltpu.async_copy(tab_hbm.at[idx_v.at[g]],
                                 rows_v.at[pl.ds(g * 128, 128)], sem)
                for g in range(G)
            ]
            for cp in cps:
                cp.wait()

            def row_body(b, _):
                i0 = b * NNZ
                acc0 = rows_v[i0, pl.ds(0, 16)]
                acc1 = rows_v[i0, pl.ds(16, 16)]
                for j in range(1, NNZ):
                    acc0 = acc0 + rows_v[i0 + j, pl.ds(0, 16)]
                    acc1 = acc1 + rows_v[i0 + j, pl.ds(16, 16)]
                out_v[b, pl.ds(0, 16)] = acc0
                out_v[b, pl.ds(16, 16)] = acc1
                return ()

            lax.fori_loop(0, CB, row_body, ())
            pltpu.sync_copy(out_v, out_hbm.at[pl.ds(row0, CB)])
            return ()

        lax.fori_loop(0, CHUNKS, chunk_body, ())

    @pl.when(wid < NS)
    def _():
        run(u_hbm, wu_hbm, p_hbm, wid * ROWS_PER_W)

    @pl.when(wid >= NS)
    def _():
        run(v_hbm, wv_hbm, q_hbm, (wid - NS) * ROWS_PER_W)


@functools.partial(
    pl.kernel,
    out_type=(
        jax.ShapeDtypeStruct((B, K), jnp.float32),
        jax.ShapeDtypeStruct((B, K), jnp.float32),
    ),
    mesh=plsc.VectorSubcoreMesh(core_axis_name="c", subcore_axis_name="s",
                                num_cores=NC, num_subcores=NS),
    scratch_types=[
        pltpu.VMEM((G, 128), jnp.int32),
        pltpu.VMEM((CB * NNZ, K), jnp.float32),
        pltpu.VMEM((CB, K), jnp.float32),
        pltpu.SemaphoreType.DMA,
    ],
)
def _tower(u_hbm, v_hbm, wu_hbm, wv_hbm, p_hbm, q_hbm, idx_v, rows_v, out_v,
           sem):
    _tower_body(u_hbm, v_hbm, wu_hbm, wv_hbm, p_hbm, q_hbm,
                idx_v, rows_v, out_v, sem)


def kernel(U, V, Wu, Wv):
    u2 = U.astype(jnp.int32).reshape(B * NNZ // 128, 128)
    v2 = V.astype(jnp.int32).reshape(B * NNZ // 128, 128)
    return _tower(u2, v2, Wu, Wv)


# speedup vs baseline: 1.8028x; 1.8028x over previous
import functools

import jax
import jax.numpy as jnp
from jax import lax
from jax.experimental import pallas as pl
from jax.experimental.pallas import tpu as pltpu
from jax.experimental.pallas import tpu_sc as plsc

B = 16384
NNZ = 26
K = 32
NC = 2
NS = 16
NW = NC * NS
CB = 64
ROWS_PER_W = B // NS
CHUNKS = ROWS_PER_W // CB
G = CB * NNZ // 128


def _tower_body(u_hbm, v_hbm, wu_hbm, wv_hbm, p_hbm, q_hbm,
                idx_v, rows_v, out_v, sem):
    wid = lax.axis_index("s") * NC + lax.axis_index("c")

    def run(idx_hbm, tab_hbm, out_hbm, base_row):
        def chunk_body(ci, _):
            row0 = base_row + ci * CB
            pltpu.sync_copy(idx_hbm.at[pl.ds(row0 * NNZ, CB * NNZ)], idx_v)
            cps = [
                pltpu.async_copy(tab_hbm.at[idx_v.at[pl.ds(g * 128, 128)]],
                                 rows_v.at[pl.ds(g * 128, 128)], sem)
                for g in range(G)
            ]
            for cp in cps:
                cp.wait()

            def row_body(b, _):
                i0 = b * NNZ
                acc0 = rows_v[i0, pl.ds(0, 16)]
                acc1 = rows_v[i0, pl.ds(16, 16)]
                for j in range(1, NNZ):
                    acc0 = acc0 + rows_v[i0 + j, pl.ds(0, 16)]
                    acc1 = acc1 + rows_v[i0 + j, pl.ds(16, 16)]
                out_v[b, pl.ds(0, 16)] = acc0
                out_v[b, pl.ds(16, 16)] = acc1
                return ()

            lax.fori_loop(0, CB, row_body, ())
            pltpu.sync_copy(out_v, out_hbm.at[pl.ds(row0, CB)])
            return ()

        lax.fori_loop(0, CHUNKS, chunk_body, ())

    @pl.when(wid < NS)
    def _():
        run(u_hbm, wu_hbm, p_hbm, wid * ROWS_PER_W)

    @pl.when(wid >= NS)
    def _():
        run(v_hbm, wv_hbm, q_hbm, (wid - NS) * ROWS_PER_W)


@functools.partial(
    pl.kernel,
    out_type=(
        jax.ShapeDtypeStruct((B, K), jnp.float32),
        jax.ShapeDtypeStruct((B, K), jnp.float32),
    ),
    mesh=plsc.VectorSubcoreMesh(core_axis_name="c", subcore_axis_name="s",
                                num_cores=NC, num_subcores=NS),
    scratch_types=[
        pltpu.VMEM((CB * NNZ,), jnp.int32),
        pltpu.VMEM((CB * NNZ, K), jnp.float32),
        pltpu.VMEM((CB, K), jnp.float32),
        pltpu.SemaphoreType.DMA,
    ],
    compiler_params=pltpu.CompilerParams(use_tc_tiling_on_sc=False),
)
def _tower(u_hbm, v_hbm, wu_hbm, wv_hbm, p_hbm, q_hbm, idx_v, rows_v, out_v,
           sem):
    _tower_body(u_hbm, v_hbm, wu_hbm, wv_hbm, p_hbm, q_hbm,
                idx_v, rows_v, out_v, sem)


def kernel(U, V, Wu, Wv):
    u1 = U.astype(jnp.int32).reshape(B * NNZ)
    v1 = V.astype(jnp.int32).reshape(B * NNZ)
    return _tower(u1, v1, Wu, Wv)
